# baseline (speedup 1.0000x reference)
import functools

import jax
import jax.numpy as jnp
from jax import lax
from jax.experimental import pallas as pl
from jax.experimental.pallas import tpu as pltpu
from jax.experimental.pallas import tpu_sc as plsc

N_TOKENS = 8192
D_MODEL = 4096
GBLK = 512
LANES = 16
SLICES = D_MODEL // LANES
NC = 2
NS = 16
NW = NC * NS
ROWS_PER_W = N_TOKENS // NW
CHUNK = 4
N_CHUNKS = ROWS_PER_W // CHUNK


def _gate_kernel(x_ref, wg_ref, w_ref):
    logits = jnp.dot(x_ref[...], wg_ref[...])
    score = jax.nn.softmax(logits, axis=-1)
    s0 = score[:, 0:1]
    s1 = score[:, 1:2]
    take0 = s0 >= s1
    w0 = jnp.where(take0, s0, 0.0)
    wt = jnp.where(take0, s0, s1)
    w_ref[...] = jnp.concatenate([w0.T, wt.T], axis=0)


@jax.jit
def _gate(x, W_gate):
    return pl.pallas_call(
        _gate_kernel,
        grid=(N_TOKENS // GBLK,),
        in_specs=[
            pl.BlockSpec((GBLK, D_MODEL), lambda i: (i, 0)),
            pl.BlockSpec((D_MODEL, 2), lambda i: (0, 0)),
        ],
        out_specs=pl.BlockSpec((2, GBLK), lambda i: (0, i)),
        out_shape=jax.ShapeDtypeStruct((2, N_TOKENS), jnp.float32),
    )(x, W_gate)


_GATHER_DNUMS = lax.GatherDimensionNumbers(
    offset_dims=(), collapsed_slice_dims=(0,), start_index_map=(0,)
)


def _bcast(w_v, rr):
    grp = (rr // LANES) * LANES
    lane = rr - grp
    wgrp = w_v[pl.ds(grp, LANES)]
    idx = jnp.full((LANES,), lane, jnp.int32)
    return lax.gather(
        wgrp, idx[:, None], _GATHER_DNUMS, slice_sizes=(1,),
        mode=lax.GatherScatterMode.PROMISE_IN_BOUNDS,
    )


def _sc_body(x_hbm, w_hbm, x0_hbm, x1_hbm, xo_hbm, w0_v, wt_v, x_v, x0_v, x1_v, xo_v):
    wid = lax.axis_index("s") * NC + lax.axis_index("c")
    row0 = wid * ROWS_PER_W
    pltpu.sync_copy(w_hbm.at[pl.ds(row0, ROWS_PER_W)], w0_v)
    pltpu.sync_copy(w_hbm.at[pl.ds(N_TOKENS + row0, ROWS_PER_W)], wt_v)

    def chunk_body(c, carry):
        base = row0 + c * CHUNK
        pltpu.sync_copy(x_hbm.at[pl.ds(base, CHUNK)], x_v)
        for r in range(CHUNK):
            rr = c * CHUNK + r
            w0v = _bcast(w0_v, rr)
            wtv = _bcast(wt_v, rr)
            w1v = wtv - w0v

            def scale_body(i, inner):
                o = i * LANES
                xs = x_v[r, pl.ds(o, LANES)]
                x0_v[r, pl.ds(o, LANES)] = xs * w0v
                x1_v[r, pl.ds(o, LANES)] = xs * w1v
                xo_v[r, pl.ds(o, LANES)] = xs * wtv
                return inner

            lax.fori_loop(0, SLICES, scale_body, 0)
        pltpu.sync_copy(x0_v, x0_hbm.at[pl.ds(base, CHUNK)])
        pltpu.sync_copy(x1_v, x1_hbm.at[pl.ds(base, CHUNK)])
        pltpu.sync_copy(xo_v, xo_hbm.at[pl.ds(base, CHUNK)])
        return carry

    lax.fori_loop(0, N_CHUNKS, chunk_body, 0)


_OUT = jax.ShapeDtypeStruct((N_TOKENS, D_MODEL), jnp.float32)

_sc_scale = functools.partial(
    pl.kernel,
    mesh=plsc.VectorSubcoreMesh(core_axis_name="c", subcore_axis_name="s"),
    out_type=[_OUT, _OUT, _OUT],
    scratch_types=[
        pltpu.VMEM((ROWS_PER_W,), jnp.float32),
        pltpu.VMEM((ROWS_PER_W,), jnp.float32),
        pltpu.VMEM((CHUNK, D_MODEL), jnp.float32),
        pltpu.VMEM((CHUNK, D_MODEL), jnp.float32),
        pltpu.VMEM((CHUNK, D_MODEL), jnp.float32),
        pltpu.VMEM((CHUNK, D_MODEL), jnp.float32),
    ],
)(_sc_body)


@jax.jit
def kernel(x, W_gate):
    w = _gate(x, W_gate).reshape(-1)
    out = _sc_scale(x, w)
    return (out[0], out[1], out[2])


# speedup vs baseline: 1.6556x; 1.6556x over previous
import functools

import jax
import jax.numpy as jnp
from jax import lax
from jax.experimental import pallas as pl
from jax.experimental.pallas import tpu as pltpu
from jax.experimental.pallas import tpu_sc as plsc

N_TOKENS = 8192
D_MODEL = 4096
GBLK = 512
LANES = 16
SLICES = D_MODEL // LANES
NC = 2
NS = 16
NW = NC * NS
ROWS_PER_W = N_TOKENS // NW
CHUNK = 2
N_CHUNKS = ROWS_PER_W // CHUNK


def _gate_kernel(x_ref, wg_ref, w_ref):
    logits = jnp.dot(x_ref[...], wg_ref[...])
    score = jax.nn.softmax(logits, axis=-1)
    s0 = score[:, 0:1]
    s1 = score[:, 1:2]
    take0 = s0 >= s1
    w0 = jnp.where(take0, s0, 0.0)
    wt = jnp.where(take0, s0, s1)
    w_ref[...] = jnp.concatenate([w0.T, wt.T], axis=0)


@jax.jit
def _gate(x, W_gate):
    return pl.pallas_call(
        _gate_kernel,
        grid=(N_TOKENS // GBLK,),
        in_specs=[
            pl.BlockSpec((GBLK, D_MODEL), lambda i: (i, 0)),
            pl.BlockSpec((D_MODEL, 2), lambda i: (0, 0)),
        ],
        out_specs=pl.BlockSpec((2, GBLK), lambda i: (0, i)),
        out_shape=jax.ShapeDtypeStruct((2, N_TOKENS), jnp.float32),
    )(x, W_gate)


_GATHER_DNUMS = lax.GatherDimensionNumbers(
    offset_dims=(), collapsed_slice_dims=(0,), start_index_map=(0,)
)


def _bcast(w_v, rr):
    grp = (rr // LANES) * LANES
    lane = rr - grp
    wgrp = w_v[pl.ds(grp, LANES)]
    idx = jnp.full((LANES,), lane, jnp.int32)
    return lax.gather(
        wgrp, idx[:, None], _GATHER_DNUMS, slice_sizes=(1,),
        mode=lax.GatherScatterMode.PROMISE_IN_BOUNDS,
    )


def _sc_body(x_hbm, w_hbm, x0_hbm, x1_hbm, xo_hbm,
             w0_v, wt_v,
             xa_v, xb_v, x0a_v, x0b_v, x1a_v, x1b_v, xoa_v, xob_v,
             in_a, in_b, out_a, out_b):
    wid = lax.axis_index("s") * NC + lax.axis_index("c")
    row0 = wid * ROWS_PER_W
    pltpu.sync_copy(w_hbm.at[pl.ds(row0, ROWS_PER_W)], w0_v)
    pltpu.sync_copy(w_hbm.at[pl.ds(N_TOKENS + row0, ROWS_PER_W)], wt_v)

    x_bufs = (xa_v, xb_v)
    o_bufs = ((x0a_v, x1a_v, xoa_v), (x0b_v, x1b_v, xob_v))
    in_sems = (in_a, in_b)
    out_sems = (out_a, out_b)

    def in_copy(cc, b):
        return pltpu.make_async_copy(
            x_hbm.at[pl.ds(row0 + cc * CHUNK, CHUNK)], x_bufs[b], in_sems[b])

    def out_copies(cc, b):
        base = row0 + cc * CHUNK
        o0, o1, oo = o_bufs[b]
        return (
            pltpu.make_async_copy(o0, x0_hbm.at[pl.ds(base, CHUNK)], out_sems[b]),
            pltpu.make_async_copy(o1, x1_hbm.at[pl.ds(base, CHUNK)], out_sems[b]),
            pltpu.make_async_copy(oo, xo_hbm.at[pl.ds(base, CHUNK)], out_sems[b]),
        )

    in_copy(0, 0).start()
    in_copy(1, 1).start()

    def pair_body(g, carry):
        for b in range(2):
            cc = g * 2 + b
            in_copy(cc, b).wait()
            @pl.when(cc >= 2)
            def _():
                for cp in out_copies(cc - 2, b):
                    cp.wait()
            x_v = x_bufs[b]
            o0, o1, oo = o_bufs[b]
            for r in range(CHUNK):
                rr = cc * CHUNK + r
                w0v = _bcast(w0_v, rr)
                wtv = _bcast(wt_v, rr)
                w1v = wtv - w0v

                def scale_body(i, inner):
                    o = i * LANES
                    xs = x_v[r, pl.ds(o, LANES)]
                    o0[r, pl.ds(o, LANES)] = xs * w0v
                    o1[r, pl.ds(o, LANES)] = xs * w1v
                    oo[r, pl.ds(o, LANES)] = xs * wtv
                    return inner

                lax.fori_loop(0, SLICES, scale_body, 0)
            for cp in out_copies(cc, b):
                cp.start()

            @pl.when(cc + 2 < N_CHUNKS)
            def _():
                in_copy(cc + 2, b).start()
        return carry

    lax.fori_loop(0, N_CHUNKS // 2, pair_body, 0)
    for b in range(2):
        for cp in out_copies(N_CHUNKS - 2 + b, b):
            cp.wait()


_OUT = jax.ShapeDtypeStruct((N_TOKENS, D_MODEL), jnp.float32)

_sc_scale = functools.partial(
    pl.kernel,
    mesh=plsc.VectorSubcoreMesh(core_axis_name="c", subcore_axis_name="s"),
    out_type=[_OUT, _OUT, _OUT],
    scratch_types=[
        pltpu.VMEM((ROWS_PER_W,), jnp.float32),
        pltpu.VMEM((ROWS_PER_W,), jnp.float32),
    ] + [pltpu.VMEM((CHUNK, D_MODEL), jnp.float32)] * 8 + [
        pltpu.SemaphoreType.DMA,
        pltpu.SemaphoreType.DMA,
        pltpu.SemaphoreType.DMA,
        pltpu.SemaphoreType.DMA,
    ],
)(_sc_body)


@jax.jit
def kernel(x, W_gate):
    w = _gate(x, W_gate).reshape(-1)
    out = _sc_scale(x, w)
    return (out[0], out[1], out[2])


# speedup vs baseline: 3.2933x; 1.9892x over previous
import jax
import jax.numpy as jnp
from jax.experimental import pallas as pl

N_TOKENS = 8192
D_MODEL = 4096
BLK = 256


def _router_kernel(x_ref, wg_ref, x0_ref, x1_ref, xout_ref):
    x = x_ref[...]
    logits = jnp.dot(x, wg_ref[...])
    score = jax.nn.softmax(logits, axis=-1)
    s0 = score[:, 0:1]
    s1 = score[:, 1:2]
    take0 = s0 >= s1
    w0 = jnp.where(take0, s0, 0.0)
    w1 = jnp.where(take0, 0.0, s1)
    x0_ref[...] = x * w0
    x1_ref[...] = x * w1
    xout_ref[...] = x * (w0 + w1)


@jax.jit
def kernel(x, W_gate):
    grid = (N_TOKENS // BLK,)
    out = pl.pallas_call(
        _router_kernel,
        grid=grid,
        in_specs=[
            pl.BlockSpec((BLK, D_MODEL), lambda i: (i, 0)),
            pl.BlockSpec((D_MODEL, 2), lambda i: (0, 0)),
        ],
        out_specs=[
            pl.BlockSpec((BLK, D_MODEL), lambda i: (i, 0)),
            pl.BlockSpec((BLK, D_MODEL), lambda i: (i, 0)),
            pl.BlockSpec((BLK, D_MODEL), lambda i: (i, 0)),
        ],
        out_shape=[
            jax.ShapeDtypeStruct((N_TOKENS, D_MODEL), x.dtype),
            jax.ShapeDtypeStruct((N_TOKENS, D_MODEL), x.dtype),
            jax.ShapeDtypeStruct((N_TOKENS, D_MODEL), x.dtype),
        ],
    )(x, W_gate)
    return (out[0], out[1], out[2])
